# baseline (speedup 1.0000x reference)
import functools

import jax
import jax.numpy as jnp
from jax import lax
from jax.experimental import pallas as pl
from jax.experimental.pallas import tpu as pltpu
from jax.experimental.pallas import tpu_sc as plsc

N = 10000
E = 320000
D = 128
NCLS = 10

NC = 2
NS = 16
NW = NC * NS

CS = 50
CHW = 200
QW = 40
NB = 4
NPAD = 10240
ROWS_PER_TILE = NPAD // NS
DEGW = NPAD // NS

_mesh = plsc.VectorSubcoreMesh(
    core_axis_name="c", subcore_axis_name="s", num_cores=NC, num_subcores=NS
)


@functools.partial(
    pl.kernel,
    out_type=jax.ShapeDtypeStruct((NC, NPAD), jnp.float32),
    mesh=_mesh,
    scratch_types=[
        pltpu.VMEM((CHW, CS), jnp.int32),
        pltpu.VMEM((CS,), jnp.float32),
        pltpu.VMEM((DEGW,), jnp.float32),
        pltpu.VMEM_SHARED((NPAD,), jnp.float32),
    ],
)
def _sc_deg(dst_hbm, degp_hbm, didx_v, ones_v, zero_v, acc_sh):
    c = lax.axis_index("c")
    s = lax.axis_index("s")
    w = c * NS + s
    pltpu.sync_copy(dst_hbm.at[w], didx_v)

    def fill(i, carry):
        ones_v[pl.ds(i * 16, 16)] = jnp.ones((16,), jnp.float32)
        return carry

    lax.fori_loop(0, CS // 16, fill, 0)
    ones_v[pl.ds(CS - 16, 16)] = jnp.ones((16,), jnp.float32)

    def zfill(i, carry):
        zero_v[pl.ds(i * 16, 16)] = jnp.zeros((16,), jnp.float32)
        return carry

    lax.fori_loop(0, DEGW // 16, zfill, 0)
    pltpu.sync_copy(zero_v, acc_sh.at[pl.ds(s * DEGW, DEGW)])
    plsc.subcore_barrier()

    def body(j, carry):
        pltpu.sync_copy(ones_v, acc_sh.at[didx_v.at[j]], add=True)
        return carry

    lax.fori_loop(0, CHW, body, 0)
    plsc.subcore_barrier()
    pltpu.sync_copy(acc_sh.at[pl.ds(s * DEGW, DEGW)],
                    degp_hbm.at[c, pl.ds(s * DEGW, DEGW)])


@functools.partial(
    pl.kernel,
    out_type=jax.ShapeDtypeStruct((NC, NPAD, D), jnp.float32),
    mesh=_mesh,
    scratch_types=[
        pltpu.VMEM((QW, CS), jnp.int32),
        pltpu.VMEM((QW, CS), jnp.int32),
        pltpu.VMEM((CS, D), jnp.float32),
        pltpu.VMEM((CS, D), jnp.float32),
        pltpu.VMEM((CS, D), jnp.float32),
        pltpu.VMEM((CS, D), jnp.float32),
        pltpu.VMEM_SHARED((NPAD, D), jnp.float32),
        pltpu.SemaphoreType.DMA,
        pltpu.SemaphoreType.DMA,
        pltpu.SemaphoreType.DMA,
        pltpu.SemaphoreType.DMA,
    ],
)
def _sc_scatter(y_hbm, src_hbm, dst_hbm, out_hbm,
                sidx_v, didx_v, rows0_v, rows1_v, rows2_v, rows3_v, acc_sh,
                sem0, sem1, sem2, sem3):
    c = lax.axis_index("c")
    s = lax.axis_index("s")
    w = c * NS + s
    r0 = s * ROWS_PER_TILE

    def zfill(i, carry):
        rows0_v[i, pl.ds(0, 16)] = jnp.zeros((16,), jnp.float32)
        rows0_v[i, pl.ds(16, 16)] = jnp.zeros((16,), jnp.float32)
        rows0_v[i, pl.ds(32, 16)] = jnp.zeros((16,), jnp.float32)
        rows0_v[i, pl.ds(48, 16)] = jnp.zeros((16,), jnp.float32)
        rows0_v[i, pl.ds(64, 16)] = jnp.zeros((16,), jnp.float32)
        rows0_v[i, pl.ds(80, 16)] = jnp.zeros((16,), jnp.float32)
        rows0_v[i, pl.ds(96, 16)] = jnp.zeros((16,), jnp.float32)
        rows0_v[i, pl.ds(112, 16)] = jnp.zeros((16,), jnp.float32)
        return carry

    lax.fori_loop(0, 40, zfill, 0)
    for k in range(16):
        pltpu.sync_copy(rows0_v.at[pl.ds(0, 40)],
                        acc_sh.at[pl.ds(r0 + 40 * k, 40)])
    plsc.subcore_barrier()

    rows = (rows0_v, rows1_v, rows2_v, rows3_v)
    sems = (sem0, sem1, sem2, sem3)
    for q in range(CHW // QW):
        pltpu.sync_copy(src_hbm.at[w, q], sidx_v)
        pltpu.sync_copy(dst_hbm.at[w, q], didx_v)
        for b in range(NB):
            pltpu.async_copy(y_hbm.at[sidx_v.at[b]], rows[b], sems[b])

        def body(t, carry):
            j0 = t * NB
            for b in range(NB):
                jb = j0 + b
                pltpu.make_async_copy(
                    y_hbm.at[sidx_v.at[jb]], rows[b], sems[b]).wait()
                pltpu.sync_copy(rows[b], acc_sh.at[didx_v.at[jb]], add=True)

                @pl.when(jb + NB < QW)
                def _():
                    pltpu.async_copy(
                        y_hbm.at[sidx_v.at[jb + NB]], rows[b], sems[b])

            return carry

        lax.fori_loop(0, QW // NB, body, 0)

    plsc.subcore_barrier()
    pltpu.sync_copy(acc_sh.at[pl.ds(r0, ROWS_PER_TILE)],
                    out_hbm.at[c, pl.ds(r0, ROWS_PER_TILE)])


R = 2000


def _tc1_body(degp_ref, x_ref, w_ref, y_ref, dis_ref):
    deg = degp_ref[:, 0:1] + degp_ref[:, 1:2] + 1.0
    dis = lax.rsqrt(deg)
    xw = jnp.dot(x_ref[...], w_ref[...], preferred_element_type=jnp.float32)
    y_ref[...] = xw * dis
    dis_ref[...] = dis


def _tc1(degp_t, x, W1):
    return pl.pallas_call(
        _tc1_body,
        grid=(N // R,),
        in_specs=[
            pl.BlockSpec((R, 2), lambda i: (i, 0)),
            pl.BlockSpec((R, D), lambda i: (i, 0)),
            pl.BlockSpec((D, D), lambda i: (0, 0)),
        ],
        out_specs=[
            pl.BlockSpec((R, D), lambda i: (i, 0)),
            pl.BlockSpec((R, 1), lambda i: (i, 0)),
        ],
        out_shape=[
            jax.ShapeDtypeStruct((N, D), jnp.float32),
            jax.ShapeDtypeStruct((N, 1), jnp.float32),
        ],
    )(degp_t, x, W1)


def _tc2_body(sp_ref0, sp_ref1, y_ref, dis_ref, b_ref, w_ref, y2_ref):
    dis = dis_ref[...]
    h = (sp_ref0[0] + sp_ref1[0] + y_ref[...]) * dis + b_ref[...]
    h = jnp.maximum(h, 0.0)
    y2_ref[...] = jnp.dot(h, w_ref[...], preferred_element_type=jnp.float32) * dis


def _tc2(sp, y, dis, b, W2):
    return pl.pallas_call(
        _tc2_body,
        grid=(N // R,),
        in_specs=[
            pl.BlockSpec((1, R, D), lambda i: (0, i, 0)),
            pl.BlockSpec((1, R, D), lambda i: (1, i, 0)),
            pl.BlockSpec((R, D), lambda i: (i, 0)),
            pl.BlockSpec((R, 1), lambda i: (i, 0)),
            pl.BlockSpec((1, D), lambda i: (0, 0)),
            pl.BlockSpec((D, D), lambda i: (0, 0)),
        ],
        out_specs=pl.BlockSpec((R, D), lambda i: (i, 0)),
        out_shape=jax.ShapeDtypeStruct((N, D), jnp.float32),
    )(sp, sp, y, dis, b, W2)


def _tc3_body(sp_ref0, sp_ref1, y_ref, dis_ref, b_ref, wfc_ref, bfc_ref, out_ref):
    dis = dis_ref[...]
    h = (sp_ref0[0] + sp_ref1[0] + y_ref[...]) * dis + b_ref[...]
    h = jnp.maximum(h, 0.0)
    out_ref[...] = (
        lax.dot_general(wfc_ref[...], h, (((0,), (1,)), ((), ())),
                        preferred_element_type=jnp.float32)
        + bfc_ref[...]
    )


def _tc3(sp, y, dis, b, Wfc, bfc):
    return pl.pallas_call(
        _tc3_body,
        grid=(1,),
        in_specs=[
            pl.BlockSpec((1, N, D), lambda i: (0, 0, 0)),
            pl.BlockSpec((1, N, D), lambda i: (1, 0, 0)),
            pl.BlockSpec((N, D), lambda i: (0, 0)),
            pl.BlockSpec((N, 1), lambda i: (0, 0)),
            pl.BlockSpec((1, D), lambda i: (0, 0)),
            pl.BlockSpec((D, NCLS), lambda i: (0, 0)),
            pl.BlockSpec((NCLS, 1), lambda i: (0, 0)),
        ],
        out_specs=pl.BlockSpec((NCLS, N), lambda i: (0, 0)),
        out_shape=jax.ShapeDtypeStruct((NCLS, N), jnp.float32),
    )(sp, sp, y, dis, b, Wfc, bfc)


def kernel(x, edge_index, W1, b1, W2, b2, Wfc, bfc):
    src4 = edge_index[0].reshape(NW, CHW // QW, QW, CS)
    dst4 = edge_index[1].reshape(NW, CHW // QW, QW, CS)
    dst3 = edge_index[1].reshape(NW, CHW, CS)

    degp = _sc_deg(dst3)
    degp_t = degp[:, :N].T

    y1, dis = _tc1(degp_t, x, W1)
    sp1 = _sc_scatter(y1, src4, dst4)
    y2 = _tc2(sp1, y1, dis, b1.reshape(1, D), W2)
    sp2 = _sc_scatter(y2, src4, dst4)
    out_t = _tc3(sp2, y2, dis, b2.reshape(1, D), Wfc, bfc.reshape(NCLS, 1))
    return out_t.T


# speedup vs baseline: 1.0507x; 1.0507x over previous
import functools

import jax
import jax.numpy as jnp
from jax import lax
from jax.experimental import pallas as pl
from jax.experimental.pallas import tpu as pltpu
from jax.experimental.pallas import tpu_sc as plsc

N = 10000
E = 320000
D = 128
NCLS = 10

NC = 2
NS = 16
NW = NC * NS

CS = 100
CHW = 100
QW = 25
NPAD = 10240
ROWS_PER_TILE = NPAD // NS
DEGW = NPAD // NS

_mesh = plsc.VectorSubcoreMesh(
    core_axis_name="c", subcore_axis_name="s", num_cores=NC, num_subcores=NS
)


@functools.partial(
    pl.kernel,
    out_type=jax.ShapeDtypeStruct((NC, NPAD), jnp.float32),
    mesh=_mesh,
    scratch_types=[
        pltpu.VMEM((CHW, CS), jnp.int32),
        pltpu.VMEM((CS,), jnp.float32),
        pltpu.VMEM((DEGW,), jnp.float32),
        pltpu.VMEM_SHARED((NPAD,), jnp.float32),
    ],
)
def _sc_deg(dst_hbm, degp_hbm, didx_v, ones_v, zero_v, acc_sh):
    c = lax.axis_index("c")
    s = lax.axis_index("s")
    w = c * NS + s
    pltpu.sync_copy(dst_hbm.at[w], didx_v)

    def fill(i, carry):
        ones_v[pl.ds(i * 16, 16)] = jnp.ones((16,), jnp.float32)
        return carry

    lax.fori_loop(0, CS // 16, fill, 0)
    ones_v[pl.ds(CS - 16, 16)] = jnp.ones((16,), jnp.float32)

    def zfill(i, carry):
        zero_v[pl.ds(i * 16, 16)] = jnp.zeros((16,), jnp.float32)
        return carry

    lax.fori_loop(0, DEGW // 16, zfill, 0)
    pltpu.sync_copy(zero_v, acc_sh.at[pl.ds(s * DEGW, DEGW)])
    plsc.subcore_barrier()

    def body(j, carry):
        pltpu.sync_copy(ones_v, acc_sh.at[didx_v.at[j]], add=True)
        return carry

    lax.fori_loop(0, CHW, body, 0)
    plsc.subcore_barrier()
    pltpu.sync_copy(acc_sh.at[pl.ds(s * DEGW, DEGW)],
                    degp_hbm.at[c, pl.ds(s * DEGW, DEGW)])


@functools.partial(
    pl.kernel,
    out_type=jax.ShapeDtypeStruct((NC, NPAD, D), jnp.float32),
    mesh=_mesh,
    scratch_types=[
        pltpu.VMEM((QW, CS), jnp.int32),
        pltpu.VMEM((QW, CS), jnp.int32),
        pltpu.VMEM((CS, D), jnp.float32),
        pltpu.VMEM((CS, D), jnp.float32),
        pltpu.VMEM((CS, D), jnp.float32),
        pltpu.VMEM_SHARED((NPAD, D), jnp.float32),
        pltpu.SemaphoreType.DMA,
        pltpu.SemaphoreType.DMA,
        pltpu.SemaphoreType.DMA,
    ],
)
def _sc_scatter(y_hbm, src_hbm, dst_hbm, out_hbm,
                sidx_v, didx_v, rows0_v, rows1_v, rows2_v, acc_sh,
                sem0, sem1, sem2):
    c = lax.axis_index("c")
    s = lax.axis_index("s")
    w = c * NS + s
    r0 = s * ROWS_PER_TILE

    def zfill(i, carry):
        rows0_v[i, pl.ds(0, 16)] = jnp.zeros((16,), jnp.float32)
        rows0_v[i, pl.ds(16, 16)] = jnp.zeros((16,), jnp.float32)
        rows0_v[i, pl.ds(32, 16)] = jnp.zeros((16,), jnp.float32)
        rows0_v[i, pl.ds(48, 16)] = jnp.zeros((16,), jnp.float32)
        rows0_v[i, pl.ds(64, 16)] = jnp.zeros((16,), jnp.float32)
        rows0_v[i, pl.ds(80, 16)] = jnp.zeros((16,), jnp.float32)
        rows0_v[i, pl.ds(96, 16)] = jnp.zeros((16,), jnp.float32)
        rows0_v[i, pl.ds(112, 16)] = jnp.zeros((16,), jnp.float32)
        return carry

    lax.fori_loop(0, 80, zfill, 0)
    for k in range(8):
        pltpu.sync_copy(rows0_v.at[pl.ds(0, 80)],
                        acc_sh.at[pl.ds(r0 + 80 * k, 80)])
    plsc.subcore_barrier()

    for q in range(4):
        pltpu.sync_copy(src_hbm.at[w, q], sidx_v)
        pltpu.sync_copy(dst_hbm.at[w, q], didx_v)
        pltpu.async_copy(y_hbm.at[sidx_v.at[0]], rows0_v, sem0)
        pltpu.async_copy(y_hbm.at[sidx_v.at[1]], rows1_v, sem1)
        pltpu.async_copy(y_hbm.at[sidx_v.at[2]], rows2_v, sem2)

        def body(t, carry):
            j0 = t * 3
            j1 = j0 + 1
            j2 = j0 + 2
            pltpu.make_async_copy(y_hbm.at[sidx_v.at[j0]], rows0_v, sem0).wait()
            pltpu.sync_copy(rows0_v, acc_sh.at[didx_v.at[j0]], add=True)

            @pl.when(j0 + 3 < QW)
            def _():
                pltpu.async_copy(y_hbm.at[sidx_v.at[j0 + 3]], rows0_v, sem0)

            pltpu.make_async_copy(y_hbm.at[sidx_v.at[j1]], rows1_v, sem1).wait()
            pltpu.sync_copy(rows1_v, acc_sh.at[didx_v.at[j1]], add=True)

            @pl.when(j1 + 3 < QW)
            def _():
                pltpu.async_copy(y_hbm.at[sidx_v.at[j1 + 3]], rows1_v, sem1)

            pltpu.make_async_copy(y_hbm.at[sidx_v.at[j2]], rows2_v, sem2).wait()
            pltpu.sync_copy(rows2_v, acc_sh.at[didx_v.at[j2]], add=True)

            @pl.when(j2 + 3 < QW)
            def _():
                pltpu.async_copy(y_hbm.at[sidx_v.at[j2 + 3]], rows2_v, sem2)

            return carry

        lax.fori_loop(0, QW // 3, body, 0)
        pltpu.make_async_copy(y_hbm.at[sidx_v.at[QW - 1]], rows0_v, sem0).wait()
        pltpu.sync_copy(rows0_v, acc_sh.at[didx_v.at[QW - 1]], add=True)

    plsc.subcore_barrier()
    pltpu.sync_copy(acc_sh.at[pl.ds(r0, ROWS_PER_TILE)],
                    out_hbm.at[c, pl.ds(r0, ROWS_PER_TILE)])


R = 2000


def _tc1_body(degp_ref, x_ref, w_ref, y_ref, dis_ref):
    deg = degp_ref[:, 0:1] + degp_ref[:, 1:2] + 1.0
    dis = lax.rsqrt(deg)
    xw = jnp.dot(x_ref[...], w_ref[...], preferred_element_type=jnp.float32)
    y_ref[...] = xw * dis
    dis_ref[...] = dis


def _tc1(degp_t, x, W1):
    return pl.pallas_call(
        _tc1_body,
        grid=(N // R,),
        in_specs=[
            pl.BlockSpec((R, 2), lambda i: (i, 0)),
            pl.BlockSpec((R, D), lambda i: (i, 0)),
            pl.BlockSpec((D, D), lambda i: (0, 0)),
        ],
        out_specs=[
            pl.BlockSpec((R, D), lambda i: (i, 0)),
            pl.BlockSpec((R, 1), lambda i: (i, 0)),
        ],
        out_shape=[
            jax.ShapeDtypeStruct((N, D), jnp.float32),
            jax.ShapeDtypeStruct((N, 1), jnp.float32),
        ],
    )(degp_t, x, W1)


def _tc2_body(sp_ref0, sp_ref1, y_ref, dis_ref, b_ref, w_ref, y2_ref):
    dis = dis_ref[...]
    h = (sp_ref0[0] + sp_ref1[0] + y_ref[...]) * dis + b_ref[...]
    h = jnp.maximum(h, 0.0)
    y2_ref[...] = jnp.dot(h, w_ref[...], preferred_element_type=jnp.float32) * dis


def _tc2(sp, y, dis, b, W2):
    return pl.pallas_call(
        _tc2_body,
        grid=(N // R,),
        in_specs=[
            pl.BlockSpec((1, R, D), lambda i: (0, i, 0)),
            pl.BlockSpec((1, R, D), lambda i: (1, i, 0)),
            pl.BlockSpec((R, D), lambda i: (i, 0)),
            pl.BlockSpec((R, 1), lambda i: (i, 0)),
            pl.BlockSpec((1, D), lambda i: (0, 0)),
            pl.BlockSpec((D, D), lambda i: (0, 0)),
        ],
        out_specs=pl.BlockSpec((R, D), lambda i: (i, 0)),
        out_shape=jax.ShapeDtypeStruct((N, D), jnp.float32),
    )(sp, sp, y, dis, b, W2)


def _tc3_body(sp_ref0, sp_ref1, y_ref, dis_ref, b_ref, wfc_ref, bfc_ref, out_ref):
    dis = dis_ref[...]
    h = (sp_ref0[0] + sp_ref1[0] + y_ref[...]) * dis + b_ref[...]
    h = jnp.maximum(h, 0.0)
    out_ref[...] = (
        lax.dot_general(wfc_ref[...], h, (((0,), (1,)), ((), ())),
                        preferred_element_type=jnp.float32)
        + bfc_ref[...]
    )


def _tc3(sp, y, dis, b, Wfc, bfc):
    return pl.pallas_call(
        _tc3_body,
        grid=(1,),
        in_specs=[
            pl.BlockSpec((1, N, D), lambda i: (0, 0, 0)),
            pl.BlockSpec((1, N, D), lambda i: (1, 0, 0)),
            pl.BlockSpec((N, D), lambda i: (0, 0)),
            pl.BlockSpec((N, 1), lambda i: (0, 0)),
            pl.BlockSpec((1, D), lambda i: (0, 0)),
            pl.BlockSpec((D, NCLS), lambda i: (0, 0)),
            pl.BlockSpec((NCLS, 1), lambda i: (0, 0)),
        ],
        out_specs=pl.BlockSpec((NCLS, N), lambda i: (0, 0)),
        out_shape=jax.ShapeDtypeStruct((NCLS, N), jnp.float32),
    )(sp, sp, y, dis, b, Wfc, bfc)


def kernel(x, edge_index, W1, b1, W2, b2, Wfc, bfc):
    src4 = edge_index[0].reshape(NW, CHW // QW, QW, CS)
    dst4 = edge_index[1].reshape(NW, CHW // QW, QW, CS)
    dst3 = edge_index[1].reshape(NW, CHW, CS)

    degp = _sc_deg(dst3)
    degp_t = degp[:, :N].T

    y1, dis = _tc1(degp_t, x, W1)
    sp1 = _sc_scatter(y1, src4, dst4)
    y2 = _tc2(sp1, y1, dis, b1.reshape(1, D), W2)
    sp2 = _sc_scatter(y2, src4, dst4)
    out_t = _tc3(sp2, y2, dis, b2.reshape(1, D), Wfc, bfc.reshape(NCLS, 1))
    return out_t.T
